# Initial kernel scaffold; baseline (speedup 1.0000x reference)
import jax
import jax.numpy as jnp
from jax.experimental import pallas as pl


def kernel(scores):
    raise NotImplementedError("write your pallas kernel here")


# speedup vs baseline: 19.6823x; 19.6823x over previous
import functools

import jax
import jax.numpy as jnp
from jax.experimental import pallas as pl
from jax.experimental.pallas import tpu as pltpu

N_BISECT = 16
N_MICHELOT = 3
ROWS_PER_BLOCK = 8


def _sparsemax_block(x_ref, o_ref):
    x = x_ref[...]
    z = x - jnp.max(x, axis=-1, keepdims=True)

    lo = jnp.full((x.shape[0], 1), -1.0, jnp.float32)
    hi = jnp.zeros((x.shape[0], 1), jnp.float32)

    def bisect(_, carry):
        lo, hi = carry
        mid = 0.5 * (lo + hi)
        f = jnp.sum(jnp.maximum(z - mid, 0.0), axis=-1, keepdims=True) - 1.0
        ge = f >= 0.0
        return jnp.where(ge, mid, lo), jnp.where(ge, hi, mid)

    lo, hi = jax.lax.fori_loop(0, N_BISECT, bisect, (lo, hi))

    def michelot(_, tau):
        m = z > tau
        k = jnp.sum(m.astype(jnp.float32), axis=-1, keepdims=True)
        s = jnp.sum(jnp.where(m, z, 0.0), axis=-1, keepdims=True)
        return (s - 1.0) / k

    tau = jax.lax.fori_loop(0, N_MICHELOT, michelot, lo)
    o_ref[...] = jnp.maximum(z - tau, 0.0)


@jax.jit
def kernel(scores):
    rows, n = scores.shape
    grid = rows // ROWS_PER_BLOCK
    return pl.pallas_call(
        _sparsemax_block,
        grid=(grid,),
        in_specs=[pl.BlockSpec((ROWS_PER_BLOCK, n), lambda i: (i, 0))],
        out_specs=pl.BlockSpec((ROWS_PER_BLOCK, n), lambda i: (i, 0)),
        out_shape=jax.ShapeDtypeStruct((rows, n), jnp.float32),
    )(scores)


# speedup vs baseline: 26.4187x; 1.3423x over previous
import jax
import jax.numpy as jnp
from jax.experimental import pallas as pl

N_BISECT = 10
N_MICHELOT = 2
ROWS_PER_BLOCK = 32
CHUNKS = 8


def _rsum(v):
    r, n = v.shape
    return jnp.sum(jnp.sum(v.reshape(r, CHUNKS, n // CHUNKS), axis=-1),
                   axis=-1, keepdims=True)


def _sparsemax_block(x_ref, o_ref):
    x = x_ref[...]
    z = x - jnp.max(x, axis=-1, keepdims=True)

    lo = jnp.full((x.shape[0], 1), -1.0, jnp.float32)
    hi = jnp.zeros((x.shape[0], 1), jnp.float32)

    for _ in range(N_BISECT):
        mid = 0.5 * (lo + hi)
        f = _rsum(jnp.maximum(z - mid, 0.0)) - 1.0
        ge = f >= 0.0
        lo = jnp.where(ge, mid, lo)
        hi = jnp.where(ge, hi, mid)

    tau = lo
    for _ in range(N_MICHELOT):
        m = z > tau
        k = _rsum(m.astype(jnp.float32))
        s = _rsum(jnp.where(m, z, 0.0))
        tau = (s - 1.0) / k

    o_ref[...] = jnp.maximum(z - tau, 0.0)


@jax.jit
def kernel(scores):
    rows, n = scores.shape
    grid = rows // ROWS_PER_BLOCK
    return pl.pallas_call(
        _sparsemax_block,
        grid=(grid,),
        in_specs=[pl.BlockSpec((ROWS_PER_BLOCK, n), lambda i: (i, 0))],
        out_specs=pl.BlockSpec((ROWS_PER_BLOCK, n), lambda i: (i, 0)),
        out_shape=jax.ShapeDtypeStruct((rows, n), jnp.float32),
    )(scores)


# speedup vs baseline: 36.5050x; 1.3818x over previous
import functools

import jax
import jax.numpy as jnp
from jax import lax
from jax.experimental import pallas as pl
from jax.experimental.pallas import tpu as pltpu
from jax.experimental.pallas import tpu_sc as plsc

ROWS = 128
N = 32768
L = 16
NCHUNK = N // L
NGVEC = NCHUNK // L
NW = 32
ROWS_PER_W = ROWS // NW

UCAP = 1024
CAP = 4096
NEG = -1e30

N_BISECT = 26
N_MICHELOT = 3
N_BISECT_FULL = 42

_mesh = plsc.VectorSubcoreMesh(core_axis_name="c", subcore_axis_name="s")


def _iota():
    return lax.iota(jnp.int32, L)


def _splat_f(x):
    return jnp.full((L,), x, jnp.float32)


def _sc_body(x_hbm, o_hbm, row_v, out_v, gmax_v, unit_v, cval_v, cidx_v):
    wid = lax.axis_index("s") * 2 + lax.axis_index("c")

    def zero_all(j, carry):
        out_v[pl.ds(j * L, L)] = _splat_f(0.0)
        return carry

    lax.fori_loop(0, NCHUNK, zero_all, 0)

    def do_row(i, carry):
        r = wid * ROWS_PER_W + i
        pltpu.sync_copy(x_hbm.at[r], row_v)

        def gmax_one(g, gacc):
            def acc_chunk(k, acc):
                return jnp.maximum(acc, row_v[pl.ds((g + NGVEC * k) * L, L)])

            m = lax.fori_loop(0, L, acc_chunk, _splat_f(NEG))
            gmax_v[pl.ds(g * L, L)] = m
            return jnp.maximum(gacc, m)

        gacc = lax.fori_loop(0, NGVEC, gmax_one, _splat_f(NEG))
        mx = jnp.max(gacc)
        mx_v = jnp.full((L,), mx, jnp.float32)
        thr_v = mx_v - 1.0

        def filt_units(g, offv):
            m = gmax_v[pl.ds(g * L, L)]
            sel = m > thr_v
            pos = offv + plsc.cumsum(sel.astype(jnp.int32)) - 1
            keep = sel & (pos < UCAP)
            plsc.store_scatter(unit_v, [pos], g * L + _iota(), mask=keep)
            return offv + plsc.all_reduce_population_count(sel)

        offv = lax.fori_loop(0, NGVEC, filt_units, jnp.zeros((L,), jnp.int32))
        nunit = jnp.max(offv)
        nunit_v = jnp.full((L,), nunit, jnp.int32)

        nub = (jnp.minimum(nunit, UCAP) + (L - 1)) // L

        def gather_units(b, offv):
            lane_ok = (b * L + _iota()) < nunit_v
            uvec = jnp.where(lane_ok, unit_v[pl.ds(b * L, L)], 0)

            def one_k(k, offv):
                idx = uvec + NCHUNK * k
                v = plsc.load_gather(row_v, [idx])
                sel = (v > thr_v) & lane_ok
                pos = offv + plsc.cumsum(sel.astype(jnp.int32)) - 1
                keep = sel & (pos < CAP)
                plsc.store_scatter(cval_v, [pos], v, mask=keep)
                plsc.store_scatter(cidx_v, [pos], idx, mask=keep)
                return offv + plsc.all_reduce_population_count(sel)

            return lax.fori_loop(0, L, one_k, offv)

        offv = lax.fori_loop(0, nub, gather_units, jnp.zeros((L,), jnp.int32))
        ncand = jnp.max(offv)
        ok = (ncand <= CAP) & (nunit <= UCAP)

        padpos = offv + _iota()
        plsc.store_scatter(cval_v, [padpos], _splat_f(NEG),
                           mask=padpos < CAP)

        nvc = (jnp.minimum(ncand, CAP) + (L - 1)) // L

        def _sum_splat(av):
            return jnp.full((L,), jnp.sum(av), jnp.float32)

        def solve_tau(buf, nvec):
            def bisect(_, lohi):
                lo, hi = lohi
                mid = 0.5 * (lo + hi)

                def facc(j, a):
                    return a + jnp.maximum(buf[pl.ds(j * L, L)] - mid, 0.0)

                f = _sum_splat(lax.fori_loop(0, nvec, facc, _splat_f(0.0)))
                ge = f >= 1.0
                return jnp.where(ge, mid, lo), jnp.where(ge, hi, mid)

            lo, _ = lax.fori_loop(0, N_BISECT, bisect,
                                  (mx_v - 1.0, mx_v))

            def michelot(_, tau):
                def ksacc(j, a):
                    ka, sa = a
                    v = buf[pl.ds(j * L, L)]
                    m = v > tau
                    return (ka + m.astype(jnp.float32),
                            sa + jnp.where(m, v, 0.0))

                ka, sa = lax.fori_loop(0, nvec, ksacc,
                                       (_splat_f(0.0), _splat_f(0.0)))
                return (_sum_splat(sa) - 1.0) / _sum_splat(ka)

            return lax.fori_loop(0, N_MICHELOT, michelot, lo)

        def tau_fast():
            return solve_tau(cval_v, nvc)

        def tau_full():
            def bisect(_, lohi):
                lo, hi = lohi
                mid = 0.5 * (lo + hi)

                def facc(j, a):
                    return a + jnp.maximum(row_v[pl.ds(j * L, L)] - mid, 0.0)

                f = _sum_splat(lax.fori_loop(0, NCHUNK, facc, _splat_f(0.0)))
                ge = f >= 1.0
                return jnp.where(ge, mid, lo), jnp.where(ge, hi, mid)

            lo, _ = lax.fori_loop(0, N_BISECT_FULL, bisect,
                                  (mx_v - 1.0, mx_v))
            return lo

        tau_v = lax.cond(ok, tau_fast, tau_full)

        def scatter_out(value_from):
            def one(j, carry):
                v = cval_v[pl.ds(j * L, L)]
                idx = cidx_v[pl.ds(j * L, L)]
                m = v > tau_v
                plsc.store_scatter(out_v, [idx], value_from(v), mask=m)
                return carry

            lax.fori_loop(0, nvc, one, 0)

        @pl.when(ok)
        def _():
            scatter_out(lambda v: v - tau_v)
            pltpu.sync_copy(out_v, o_hbm.at[r])
            scatter_out(lambda v: _splat_f(0.0))

        @pl.when(jnp.logical_not(ok))
        def _():
            def dense(j, carry):
                v = row_v[pl.ds(j * L, L)]
                out_v[pl.ds(j * L, L)] = jnp.maximum(v - tau_v, 0.0)
                return carry

            lax.fori_loop(0, NCHUNK, dense, 0)
            pltpu.sync_copy(out_v, o_hbm.at[r])
            lax.fori_loop(0, NCHUNK, zero_all, 0)

        return carry

    lax.fori_loop(0, ROWS_PER_W, do_row, 0)


@jax.jit
def kernel(scores):
    f = functools.partial(
        pl.kernel,
        mesh=_mesh,
        out_type=jax.ShapeDtypeStruct((ROWS, N), jnp.float32),
        compiler_params=pltpu.CompilerParams(needs_layout_passes=False),
        scratch_types=[
            pltpu.VMEM((N,), jnp.float32),
            pltpu.VMEM((N,), jnp.float32),
            pltpu.VMEM((NCHUNK,), jnp.float32),
            pltpu.VMEM((UCAP,), jnp.int32),
            pltpu.VMEM((CAP,), jnp.float32),
            pltpu.VMEM((CAP,), jnp.int32),
        ],
    )(_sc_body)
    return f(scores)


# speedup vs baseline: 40.6873x; 1.1146x over previous
import functools

import jax
import jax.numpy as jnp
from jax import lax
from jax.experimental import pallas as pl
from jax.experimental.pallas import tpu as pltpu
from jax.experimental.pallas import tpu_sc as plsc

ROWS = 128
N = 32768
L = 16
NCHUNK = N // L
NGVEC = NCHUNK // L
NW = 32
ROWS_PER_W = ROWS // NW

UCAP = 1024
CAP = 4096
NEG = -1e30

N_BISECT = 26
N_MICHELOT = 3
N_BISECT_FULL = 42

_mesh = plsc.VectorSubcoreMesh(core_axis_name="c", subcore_axis_name="s")


def _iota():
    return lax.iota(jnp.int32, L)


def _splat_f(x):
    return jnp.full((L,), x, jnp.float32)


def _sc_body(x_hbm, o_hbm, row_v, out_v, gmax_v, unit_v, cval_v, cidx_v):
    wid = lax.axis_index("s") * 2 + lax.axis_index("c")

    ZUNROLL = 8

    def zero_all(j, carry):
        for u in range(ZUNROLL):
            out_v[pl.ds((j * ZUNROLL + u) * L, L)] = _splat_f(0.0)
        return carry

    lax.fori_loop(0, NCHUNK // ZUNROLL, zero_all, 0)

    def _tree_max(vs):
        while len(vs) > 1:
            vs = [jnp.maximum(a, b) for a, b in zip(vs[::2], vs[1::2])]
        return vs[0]

    def do_row(i, carry):
        r = wid * ROWS_PER_W + i
        pltpu.sync_copy(x_hbm.at[r], row_v)

        def gmax_one(g, carry):
            vs = [row_v[pl.ds((g + NGVEC * k) * L, L)] for k in range(L)]
            gmax_v[pl.ds(g * L, L)] = _tree_max(vs)
            return carry

        lax.fori_loop(0, NGVEC, gmax_one, 0)

        def max_red(j, acc):
            vs = [gmax_v[pl.ds((j * 8 + u) * L, L)] for u in range(8)]
            return jnp.maximum(acc, _tree_max(vs))

        gacc = lax.fori_loop(0, NGVEC // 8, max_red, _splat_f(NEG))
        mx = jnp.max(gacc)
        mx_v = jnp.full((L,), mx, jnp.float32)
        thr_v = mx_v - 1.0

        def filt_units(g2, offv):
            for u in range(2):
                g = g2 * 2 + u
                m = gmax_v[pl.ds(g * L, L)]
                sel = m > thr_v
                pos = offv + plsc.cumsum(sel.astype(jnp.int32)) - 1
                keep = sel & (pos < UCAP)
                plsc.store_scatter(unit_v, [pos], g * L + _iota(), mask=keep)
                offv = offv + plsc.all_reduce_population_count(sel)
            return offv

        offv = lax.fori_loop(0, NGVEC // 2, filt_units,
                             jnp.zeros((L,), jnp.int32))
        nunit = jnp.max(offv)
        nunit_v = jnp.full((L,), nunit, jnp.int32)

        nub = (jnp.minimum(nunit, UCAP) + (L - 1)) // L

        def gather_units(b, offv):
            lane_ok = (b * L + _iota()) < nunit_v
            uvec = jnp.where(lane_ok, unit_v[pl.ds(b * L, L)], 0)

            def one_k(k, offv):
                idx = uvec + NCHUNK * k
                v = plsc.load_gather(row_v, [idx])
                sel = (v > thr_v) & lane_ok
                pos = offv + plsc.cumsum(sel.astype(jnp.int32)) - 1
                keep = sel & (pos < CAP)
                plsc.store_scatter(cval_v, [pos], v, mask=keep)
                plsc.store_scatter(cidx_v, [pos], idx, mask=keep)
                return offv + plsc.all_reduce_population_count(sel)

            return lax.fori_loop(0, L, one_k, offv)

        offv = lax.fori_loop(0, nub, gather_units, jnp.zeros((L,), jnp.int32))
        ncand = jnp.max(offv)
        ok = (ncand <= CAP) & (nunit <= UCAP)

        padpos = offv + _iota()
        plsc.store_scatter(cval_v, [padpos], _splat_f(NEG),
                           mask=padpos < CAP)

        nvc = (jnp.minimum(ncand, CAP) + (L - 1)) // L

        def _sum_splat(av):
            return jnp.full((L,), jnp.sum(av), jnp.float32)

        def solve_tau(buf, nvec):
            def bisect(_, lohi):
                lo, hi = lohi
                mid = 0.5 * (lo + hi)

                def facc(j, a):
                    return a + jnp.maximum(buf[pl.ds(j * L, L)] - mid, 0.0)

                f = _sum_splat(lax.fori_loop(0, nvec, facc, _splat_f(0.0)))
                ge = f >= 1.0
                return jnp.where(ge, mid, lo), jnp.where(ge, hi, mid)

            lo, _ = lax.fori_loop(0, N_BISECT, bisect,
                                  (mx_v - 1.0, mx_v))

            def michelot(_, tau):
                def ksacc(j, a):
                    ka, sa = a
                    v = buf[pl.ds(j * L, L)]
                    m = v > tau
                    return (ka + m.astype(jnp.float32),
                            sa + jnp.where(m, v, 0.0))

                ka, sa = lax.fori_loop(0, nvec, ksacc,
                                       (_splat_f(0.0), _splat_f(0.0)))
                return (_sum_splat(sa) - 1.0) / _sum_splat(ka)

            return lax.fori_loop(0, N_MICHELOT, michelot, lo)

        def tau_fast():
            return solve_tau(cval_v, nvc)

        def tau_full():
            def bisect(_, lohi):
                lo, hi = lohi
                mid = 0.5 * (lo + hi)

                def facc(j, a):
                    return a + jnp.maximum(row_v[pl.ds(j * L, L)] - mid, 0.0)

                f = _sum_splat(lax.fori_loop(0, NCHUNK, facc, _splat_f(0.0)))
                ge = f >= 1.0
                return jnp.where(ge, mid, lo), jnp.where(ge, hi, mid)

            lo, _ = lax.fori_loop(0, N_BISECT_FULL, bisect,
                                  (mx_v - 1.0, mx_v))
            return lo

        tau_v = lax.cond(ok, tau_fast, tau_full)

        def scatter_out(value_from):
            def one(j, carry):
                v = cval_v[pl.ds(j * L, L)]
                idx = cidx_v[pl.ds(j * L, L)]
                m = v > tau_v
                plsc.store_scatter(out_v, [idx], value_from(v), mask=m)
                return carry

            lax.fori_loop(0, nvc, one, 0)

        @pl.when(ok)
        def _():
            scatter_out(lambda v: v - tau_v)
            pltpu.sync_copy(out_v, o_hbm.at[r])
            scatter_out(lambda v: _splat_f(0.0))

        @pl.when(jnp.logical_not(ok))
        def _():
            def dense(j, carry):
                v = row_v[pl.ds(j * L, L)]
                out_v[pl.ds(j * L, L)] = jnp.maximum(v - tau_v, 0.0)
                return carry

            lax.fori_loop(0, NCHUNK, dense, 0)
            pltpu.sync_copy(out_v, o_hbm.at[r])
            lax.fori_loop(0, NCHUNK, zero_all, 0)

        return carry

    lax.fori_loop(0, ROWS_PER_W, do_row, 0)


@jax.jit
def kernel(scores):
    f = functools.partial(
        pl.kernel,
        mesh=_mesh,
        out_type=jax.ShapeDtypeStruct((ROWS, N), jnp.float32),
        compiler_params=pltpu.CompilerParams(needs_layout_passes=False),
        scratch_types=[
            pltpu.VMEM((N,), jnp.float32),
            pltpu.VMEM((N,), jnp.float32),
            pltpu.VMEM((NCHUNK,), jnp.float32),
            pltpu.VMEM((UCAP,), jnp.int32),
            pltpu.VMEM((CAP,), jnp.float32),
            pltpu.VMEM((CAP,), jnp.int32),
        ],
    )(_sc_body)
    return f(scores)


# speedup vs baseline: 47.0981x; 1.1576x over previous
import functools

import jax
import jax.numpy as jnp
from jax import lax
from jax.experimental import pallas as pl
from jax.experimental.pallas import tpu as pltpu
from jax.experimental.pallas import tpu_sc as plsc

ROWS = 128
N = 32768
L = 16
NCHUNK = N // L
NGVEC = NCHUNK // L
NW = 32
ROWS_PER_W = ROWS // NW

UCAP = 1024
CAP = 4096
NEG = -1e30
ZUNROLL = 8

N_BISECT = 12
N_MICHELOT = 3
N_BISECT_FULL = 42

_mesh = plsc.VectorSubcoreMesh(core_axis_name="c", subcore_axis_name="s")


def _iota():
    return lax.iota(jnp.int32, L)


def _splat_f(x):
    return jnp.full((L,), x, jnp.float32)


def _tree_max(vs):
    while len(vs) > 1:
        vs = [jnp.maximum(a, b) for a, b in zip(vs[::2], vs[1::2])]
    return vs[0]


def _sum_splat(av):
    return jnp.full((L,), jnp.sum(av), jnp.float32)


def _sc_body(x_hbm, o_hbm, row0_v, row1_v, out_v, gmax_v, unit_v,
             cval_v, cidx_v, sem0, sem1):
    wid = lax.axis_index("s") * 2 + lax.axis_index("c")
    base = wid * ROWS_PER_W

    def zero_all(j, carry):
        for u in range(ZUNROLL):
            out_v[pl.ds((j * ZUNROLL + u) * L, L)] = _splat_f(0.0)
        return carry

    lax.fori_loop(0, NCHUNK // ZUNROLL, zero_all, 0)

    def process(row_v, r):
        def gmax_one(g, carry):
            vs = [row_v[pl.ds((g + NGVEC * k) * L, L)] for k in range(L)]
            gmax_v[pl.ds(g * L, L)] = _tree_max(vs)
            return carry

        lax.fori_loop(0, NGVEC, gmax_one, 0)

        def max_red(j, acc):
            vs = [gmax_v[pl.ds((j * 8 + u) * L, L)] for u in range(8)]
            return jnp.maximum(acc, _tree_max(vs))

        gacc = lax.fori_loop(0, NGVEC // 8, max_red, _splat_f(NEG))
        mx_v = jnp.full((L,), jnp.max(gacc), jnp.float32)
        thr_v = mx_v - 1.0

        def filt_units(q, offv):
            ms = [gmax_v[pl.ds((q * 4 + u) * L, L)] for u in range(4)]
            sels = [m > thr_v for m in ms]
            anysel = (sels[0] | sels[1]) | (sels[2] | sels[3])
            hit = plsc.all_reduce_population_count(anysel)[0] > 0

            def slow(offv):
                for u in range(4):
                    sel = sels[u]
                    pos = offv + plsc.cumsum(sel.astype(jnp.int32)) - 1
                    keep = sel & (pos < UCAP)
                    plsc.store_scatter(unit_v, [pos],
                                       (q * 4 + u) * L + _iota(), mask=keep)
                    offv = offv + plsc.all_reduce_population_count(sel)
                return offv

            return lax.cond(hit, slow, lambda o: o, offv)

        offv = lax.fori_loop(0, NGVEC // 4, filt_units,
                             jnp.zeros((L,), jnp.int32))
        nunit = jnp.max(offv)
        nunit_v = jnp.full((L,), nunit, jnp.int32)

        nub = (jnp.minimum(nunit, UCAP) + (L - 1)) // L

        def gather_units(b, offv):
            lane_ok = (b * L + _iota()) < nunit_v
            uvec = jnp.where(lane_ok, unit_v[pl.ds(b * L, L)], 0)

            def one_k(k, offv):
                idx = uvec + NCHUNK * k
                v = plsc.load_gather(row_v, [idx])
                sel = (v > thr_v) & lane_ok
                pos = offv + plsc.cumsum(sel.astype(jnp.int32)) - 1
                keep = sel & (pos < CAP)
                plsc.store_scatter(cval_v, [pos], v, mask=keep)
                plsc.store_scatter(cidx_v, [pos], idx, mask=keep)
                return offv + plsc.all_reduce_population_count(sel)

            return lax.fori_loop(0, L, one_k, offv)

        offv = lax.fori_loop(0, nub, gather_units, jnp.zeros((L,), jnp.int32))
        ncand = jnp.max(offv)
        ok = (ncand <= CAP) & (nunit <= UCAP)

        padpos = offv + _iota()
        plsc.store_scatter(cval_v, [padpos], _splat_f(NEG),
                           mask=padpos < CAP)

        nvc = (jnp.minimum(ncand, CAP) + (L - 1)) // L

        def tau_fast():
            def bisect(_, lohi):
                lo, hi = lohi
                mid = 0.5 * (lo + hi)

                def facc(j, a):
                    return a + jnp.maximum(cval_v[pl.ds(j * L, L)] - mid, 0.0)

                f = _sum_splat(lax.fori_loop(0, nvc, facc, _splat_f(0.0)))
                ge = f >= 1.0
                return jnp.where(ge, mid, lo), jnp.where(ge, hi, mid)

            lo, _ = lax.fori_loop(0, N_BISECT, bisect, (mx_v - 1.0, mx_v))

            def michelot(_, tau):
                def ksacc(j, a):
                    ka, sa = a
                    v = cval_v[pl.ds(j * L, L)]
                    m = v > tau
                    return (ka + m.astype(jnp.float32),
                            sa + jnp.where(m, v, 0.0))

                ka, sa = lax.fori_loop(0, nvc, ksacc,
                                       (_splat_f(0.0), _splat_f(0.0)))
                return (_sum_splat(sa) - 1.0) / _sum_splat(ka)

            return lax.fori_loop(0, N_MICHELOT, michelot, lo)

        def tau_full():
            def bisect(_, lohi):
                lo, hi = lohi
                mid = 0.5 * (lo + hi)

                def facc(j, a):
                    return a + jnp.maximum(row_v[pl.ds(j * L, L)] - mid, 0.0)

                f = _sum_splat(lax.fori_loop(0, NCHUNK, facc, _splat_f(0.0)))
                ge = f >= 1.0
                return jnp.where(ge, mid, lo), jnp.where(ge, hi, mid)

            lo, _ = lax.fori_loop(0, N_BISECT_FULL, bisect,
                                  (mx_v - 1.0, mx_v))
            return lo

        tau_v = lax.cond(ok, tau_fast, tau_full)

        def scatter_out(value_from):
            def one(j, carry):
                v = cval_v[pl.ds(j * L, L)]
                idx = cidx_v[pl.ds(j * L, L)]
                m = v > tau_v
                plsc.store_scatter(out_v, [idx], value_from(v), mask=m)
                return carry

            lax.fori_loop(0, nvc, one, 0)

        @pl.when(ok)
        def _():
            scatter_out(lambda v: v - tau_v)
            pltpu.sync_copy(out_v, o_hbm.at[r])
            scatter_out(lambda v: _splat_f(0.0))

        @pl.when(jnp.logical_not(ok))
        def _():
            def dense(j, carry):
                v = row_v[pl.ds(j * L, L)]
                out_v[pl.ds(j * L, L)] = jnp.maximum(v - tau_v, 0.0)
                return carry

            lax.fori_loop(0, NCHUNK, dense, 0)
            pltpu.sync_copy(out_v, o_hbm.at[r])
            lax.fori_loop(0, NCHUNK // ZUNROLL, zero_all, 0)

    bufs = [(row0_v, sem0), (row1_v, sem1)]
    handles = [pltpu.async_copy(x_hbm.at[base], row0_v, sem0), None]
    for i in range(ROWS_PER_W):
        buf, _ = bufs[i % 2]
        handles[i % 2].wait()
        if i + 1 < ROWS_PER_W:
            nbuf, nsem = bufs[(i + 1) % 2]
            handles[(i + 1) % 2] = pltpu.async_copy(
                x_hbm.at[base + i + 1], nbuf, nsem)
        process(buf, base + i)


@jax.jit
def kernel(scores):
    f = functools.partial(
        pl.kernel,
        mesh=_mesh,
        out_type=jax.ShapeDtypeStruct((ROWS, N), jnp.float32),
        compiler_params=pltpu.CompilerParams(needs_layout_passes=False),
        scratch_types=[
            pltpu.VMEM((N,), jnp.float32),
            pltpu.VMEM((N,), jnp.float32),
            pltpu.VMEM((N,), jnp.float32),
            pltpu.VMEM((NCHUNK,), jnp.float32),
            pltpu.VMEM((UCAP,), jnp.int32),
            pltpu.VMEM((CAP,), jnp.float32),
            pltpu.VMEM((CAP,), jnp.int32),
            pltpu.SemaphoreType.DMA,
            pltpu.SemaphoreType.DMA,
        ],
    )(_sc_body)
    return f(scores)


# speedup vs baseline: 54.2248x; 1.1513x over previous
import functools

import jax
import jax.numpy as jnp
from jax import lax
from jax.experimental import pallas as pl
from jax.experimental.pallas import tpu as pltpu
from jax.experimental.pallas import tpu_sc as plsc

ROWS = 128
N = 32768
L = 16
NCHUNK = N // L
NGVEC = NCHUNK // L
NW = 32
ROWS_PER_W = ROWS // NW

UCAP = 1024
CAP = 4096
NEG = -1e30
ZUNROLL = 8

N_BISECT = 12
N_MICHELOT = 3
N_BISECT_FULL = 42

_mesh = plsc.VectorSubcoreMesh(core_axis_name="c", subcore_axis_name="s")


def _iota():
    return lax.iota(jnp.int32, L)


def _splat_f(x):
    return jnp.full((L,), x, jnp.float32)


def _tree_max(vs):
    while len(vs) > 1:
        vs = [jnp.maximum(a, b) for a, b in zip(vs[::2], vs[1::2])]
    return vs[0]


def _sum_splat(av):
    return jnp.full((L,), jnp.sum(av), jnp.float32)


def _sc_body(x_hbm, o_hbm, row0_v, row1_v, out_v, gmax_v, unit_v,
             cval0_v, cidx0_v, cval1_v, cidx1_v, sem0, sem1, osem):
    wid = lax.axis_index("s") * 2 + lax.axis_index("c")
    base = wid * ROWS_PER_W

    def zero_all(j, carry):
        for u in range(ZUNROLL):
            out_v[pl.ds((j * ZUNROLL + u) * L, L)] = _splat_f(0.0)
        return carry

    lax.fori_loop(0, NCHUNK // ZUNROLL, zero_all, 0)

    def filter_row(row_v, cval_v, cidx_v):
        def gmax_one(g2, carry):
            for u in range(2):
                g = g2 * 2 + u
                vs = [row_v[pl.ds((g + NGVEC * k) * L, L)] for k in range(L)]
                gmax_v[pl.ds(g * L, L)] = _tree_max(vs)
            return carry

        lax.fori_loop(0, NGVEC // 2, gmax_one, 0)

        def max_red(j, acc):
            vs = [gmax_v[pl.ds((j * 8 + u) * L, L)] for u in range(8)]
            return jnp.maximum(acc, _tree_max(vs))

        gacc = lax.fori_loop(0, NGVEC // 8, max_red, _splat_f(NEG))
        mx_v = jnp.full((L,), jnp.max(gacc), jnp.float32)
        thr_v = mx_v - 1.0

        def filt_units(q, offv):
            ms = [gmax_v[pl.ds((q * 4 + u) * L, L)] for u in range(4)]
            sels = [m > thr_v for m in ms]
            anysel = (sels[0] | sels[1]) | (sels[2] | sels[3])
            hit = plsc.all_reduce_population_count(anysel)[0] > 0

            def slow(offv):
                for u in range(4):
                    sel = sels[u]
                    pos = offv + plsc.cumsum(sel.astype(jnp.int32)) - 1
                    keep = sel & (pos < UCAP)
                    plsc.store_scatter(unit_v, [pos],
                                       (q * 4 + u) * L + _iota(), mask=keep)
                    offv = offv + plsc.all_reduce_population_count(sel)
                return offv

            return lax.cond(hit, slow, lambda o: o, offv)

        offv = lax.fori_loop(0, NGVEC // 4, filt_units,
                             jnp.zeros((L,), jnp.int32))
        nunit = jnp.max(offv)
        nunit_v = jnp.full((L,), nunit, jnp.int32)

        nub = (jnp.minimum(nunit, UCAP) + (L - 1)) // L

        def gather_units(b, offv):
            lane_ok = (b * L + _iota()) < nunit_v
            uvec = jnp.where(lane_ok, unit_v[pl.ds(b * L, L)], 0)

            def one_k(k, offv):
                idx = uvec + NCHUNK * k
                v = plsc.load_gather(row_v, [idx])
                sel = (v > thr_v) & lane_ok
                pos = offv + plsc.cumsum(sel.astype(jnp.int32)) - 1
                keep = sel & (pos < CAP)
                plsc.store_scatter(cval_v, [pos], v, mask=keep)
                plsc.store_scatter(cidx_v, [pos], idx, mask=keep)
                return offv + plsc.all_reduce_population_count(sel)

            return lax.fori_loop(0, L, one_k, offv)

        offv = lax.fori_loop(0, nub, gather_units, jnp.zeros((L,), jnp.int32))
        ncand = jnp.max(offv)
        ok = (ncand <= CAP) & (nunit <= UCAP)

        padpos = offv + _iota()
        plsc.store_scatter(cval_v, [padpos], _splat_f(NEG),
                           mask=padpos < CAP)

        nvc = (jnp.minimum(ncand, CAP) + (L - 1)) // L
        return ok, nvc, mx_v

    def solve_tau(row_v, cval_v, ok, nvc, mx_v):
        def tau_fast():
            def bisect(_, lohi):
                lo, hi = lohi
                mid = 0.5 * (lo + hi)

                def facc(j, a):
                    return a + jnp.maximum(cval_v[pl.ds(j * L, L)] - mid, 0.0)

                f = _sum_splat(lax.fori_loop(0, nvc, facc, _splat_f(0.0)))
                ge = f >= 1.0
                return jnp.where(ge, mid, lo), jnp.where(ge, hi, mid)

            lo, _ = lax.fori_loop(0, N_BISECT, bisect, (mx_v - 1.0, mx_v))

            def michelot(_, tau):
                def ksacc(j, a):
                    ka, sa = a
                    v = cval_v[pl.ds(j * L, L)]
                    m = v > tau
                    return (ka + m.astype(jnp.float32),
                            sa + jnp.where(m, v, 0.0))

                ka, sa = lax.fori_loop(0, nvc, ksacc,
                                       (_splat_f(0.0), _splat_f(0.0)))
                return (_sum_splat(sa) - 1.0) / _sum_splat(ka)

            return lax.fori_loop(0, N_MICHELOT, michelot, lo)

        def tau_full():
            def bisect(_, lohi):
                lo, hi = lohi
                mid = 0.5 * (lo + hi)

                def facc(j, a):
                    return a + jnp.maximum(row_v[pl.ds(j * L, L)] - mid, 0.0)

                f = _sum_splat(lax.fori_loop(0, NCHUNK, facc, _splat_f(0.0)))
                ge = f >= 1.0
                return jnp.where(ge, mid, lo), jnp.where(ge, hi, mid)

            lo, _ = lax.fori_loop(0, N_BISECT_FULL, bisect,
                                  (mx_v - 1.0, mx_v))
            return lo

        return lax.cond(ok, tau_fast, tau_full)

    def scatter_list(cval_v, cidx_v, nvc, tau_v, value_from):
        def one(j, carry):
            v = cval_v[pl.ds(j * L, L)]
            idx = cidx_v[pl.ds(j * L, L)]
            m = v > tau_v
            plsc.store_scatter(out_v, [idx], value_from(v), mask=m)
            return carry

        lax.fori_loop(0, nvc, one, 0)

    def restore_zeros(prev):
        h, cval_v, cidx_v, nvc, tau_v, ok = prev
        h.wait()

        @pl.when(ok)
        def _():
            scatter_list(cval_v, cidx_v, nvc, tau_v, lambda v: _splat_f(0.0))

        @pl.when(jnp.logical_not(ok))
        def _():
            lax.fori_loop(0, NCHUNK // ZUNROLL, zero_all, 0)

    ibufs = [(row0_v, sem0), (row1_v, sem1)]
    cbufs = [(cval0_v, cidx0_v), (cval1_v, cidx1_v)]
    handles = [pltpu.async_copy(x_hbm.at[base], row0_v, sem0), None]
    pending = None
    for i in range(ROWS_PER_W):
        row_v, _ = ibufs[i % 2]
        cval_v, cidx_v = cbufs[i % 2]
        handles[i % 2].wait()
        if i + 1 < ROWS_PER_W:
            nbuf, nsem = ibufs[(i + 1) % 2]
            handles[(i + 1) % 2] = pltpu.async_copy(
                x_hbm.at[base + i + 1], nbuf, nsem)

        ok, nvc, mx_v = filter_row(row_v, cval_v, cidx_v)
        tau_v = solve_tau(row_v, cval_v, ok, nvc, mx_v)

        if pending is not None:
            restore_zeros(pending)

        @pl.when(ok)
        def _():
            scatter_list(cval_v, cidx_v, nvc, tau_v, lambda v: v - tau_v)

        @pl.when(jnp.logical_not(ok))
        def _():
            def dense(j, carry):
                v = row_v[pl.ds(j * L, L)]
                out_v[pl.ds(j * L, L)] = jnp.maximum(v - tau_v, 0.0)
                return carry

            lax.fori_loop(0, NCHUNK, dense, 0)

        h = pltpu.async_copy(out_v, o_hbm.at[base + i], osem)
        pending = (h, cval_v, cidx_v, nvc, tau_v, ok)

    pending[0].wait()


@jax.jit
def kernel(scores):
    f = functools.partial(
        pl.kernel,
        mesh=_mesh,
        out_type=jax.ShapeDtypeStruct((ROWS, N), jnp.float32),
        compiler_params=pltpu.CompilerParams(needs_layout_passes=False),
        scratch_types=[
            pltpu.VMEM((N,), jnp.float32),
            pltpu.VMEM((N,), jnp.float32),
            pltpu.VMEM((N,), jnp.float32),
            pltpu.VMEM((NCHUNK,), jnp.float32),
            pltpu.VMEM((UCAP,), jnp.int32),
            pltpu.VMEM((CAP,), jnp.float32),
            pltpu.VMEM((CAP,), jnp.int32),
            pltpu.VMEM((CAP,), jnp.float32),
            pltpu.VMEM((CAP,), jnp.int32),
            pltpu.SemaphoreType.DMA,
            pltpu.SemaphoreType.DMA,
            pltpu.SemaphoreType.DMA,
        ],
    )(_sc_body)
    return f(scores)


# speedup vs baseline: 54.2549x; 1.0006x over previous
import functools

import jax
import jax.numpy as jnp
from jax import lax
from jax.experimental import pallas as pl
from jax.experimental.pallas import tpu as pltpu
from jax.experimental.pallas import tpu_sc as plsc

ROWS = 128
N = 32768
L = 16
NCHUNK = N // L
NGVEC = NCHUNK // L
NW = 32
ROWS_PER_W = ROWS // NW

UCAP = 1024
CAP = 4096
NEG = -1e30
ZUNROLL = 8

N_BISECT = 12
N_MICHELOT = 3
N_BISECT_FULL = 42

_mesh = plsc.VectorSubcoreMesh(core_axis_name="c", subcore_axis_name="s")


def _iota():
    return lax.iota(jnp.int32, L)


def _splat_f(x):
    return jnp.full((L,), x, jnp.float32)


def _tree_max(vs):
    while len(vs) > 1:
        vs = [jnp.maximum(a, b) for a, b in zip(vs[::2], vs[1::2])]
    return vs[0]


def _sum_splat(av):
    return jnp.full((L,), jnp.sum(av), jnp.float32)


def _sc_body(x_hbm, o_hbm, row0_v, row1_v, out_v, gmax_v, unit_v,
             cval0_v, cidx0_v, cval1_v, cidx1_v, sem0, sem1, osem):
    wid = lax.axis_index("s") * 2 + lax.axis_index("c")
    base = wid * ROWS_PER_W

    def zero_all(j, carry):
        for u in range(ZUNROLL):
            out_v[pl.ds((j * ZUNROLL + u) * L, L)] = _splat_f(0.0)
        return carry

    lax.fori_loop(0, NCHUNK // ZUNROLL, zero_all, 0)

    def filter_row(row_v, cval_v, cidx_v):
        def gmax_one(g2, carry):
            for u in range(2):
                g = g2 * 2 + u
                vs = [row_v[pl.ds((g + NGVEC * k) * L, L)] for k in range(L)]
                gmax_v[pl.ds(g * L, L)] = _tree_max(vs)
            return carry

        lax.fori_loop(0, NGVEC // 2, gmax_one, 0)

        def max_red(j, acc):
            vs = [gmax_v[pl.ds((j * 8 + u) * L, L)] for u in range(8)]
            return jnp.maximum(acc, _tree_max(vs))

        gacc = lax.fori_loop(0, NGVEC // 8, max_red, _splat_f(NEG))
        mx_v = jnp.full((L,), jnp.max(gacc), jnp.float32)
        thr_v = mx_v - 1.0

        def filt_units(q, offv):
            ms = [gmax_v[pl.ds((q * 4 + u) * L, L)] for u in range(4)]
            sels = [m > thr_v for m in ms]
            anysel = (sels[0] | sels[1]) | (sels[2] | sels[3])
            hit = plsc.all_reduce_population_count(anysel)[0] > 0

            def slow(offv):
                for u in range(4):
                    sel = sels[u]
                    pos = offv + plsc.cumsum(sel.astype(jnp.int32)) - 1
                    keep = sel & (pos < UCAP)
                    plsc.store_scatter(unit_v, [pos],
                                       (q * 4 + u) * L + _iota(), mask=keep)
                    offv = offv + plsc.all_reduce_population_count(sel)
                return offv

            return lax.cond(hit, slow, lambda o: o, offv)

        offv = lax.fori_loop(0, NGVEC // 4, filt_units,
                             jnp.zeros((L,), jnp.int32))
        nunit = jnp.max(offv)
        nunit_v = jnp.full((L,), nunit, jnp.int32)

        nub = (jnp.minimum(nunit, UCAP) + (L - 1)) // L

        def gather_units(b, offv):
            lane_ok = (b * L + _iota()) < nunit_v
            uvec = jnp.where(lane_ok, unit_v[pl.ds(b * L, L)], 0)

            def one_k(k, offv):
                idx = uvec + NCHUNK * k
                v = plsc.load_gather(row_v, [idx])
                sel = (v > thr_v) & lane_ok
                pos = offv + plsc.cumsum(sel.astype(jnp.int32)) - 1
                keep = sel & (pos < CAP)
                plsc.store_scatter(cval_v, [pos], v, mask=keep)
                plsc.store_scatter(cidx_v, [pos], idx, mask=keep)
                return offv + plsc.all_reduce_population_count(sel)

            return lax.fori_loop(0, L, one_k, offv)

        offv = lax.fori_loop(0, nub, gather_units, jnp.zeros((L,), jnp.int32))
        ncand = jnp.max(offv)
        ok = (ncand <= CAP) & (nunit <= UCAP)

        padpos = offv + _iota()
        plsc.store_scatter(cval_v, [padpos], _splat_f(NEG),
                           mask=padpos < CAP)

        nvc = (jnp.minimum(ncand, CAP) + (L - 1)) // L
        return ok, nvc, mx_v

    def solve_tau(row_v, cval_v, ok, nvc, mx_v):
        def tau_fast():
            def bisect(_, lohi):
                lo, hi = lohi
                mid = 0.5 * (lo + hi)

                def facc(j, a):
                    return a + jnp.maximum(cval_v[pl.ds(j * L, L)] - mid, 0.0)

                f = _sum_splat(lax.fori_loop(0, nvc, facc, _splat_f(0.0)))
                ge = f >= 1.0
                return jnp.where(ge, mid, lo), jnp.where(ge, hi, mid)

            lo, _ = lax.fori_loop(0, N_BISECT, bisect, (mx_v - 1.0, mx_v))

            def michelot(_, tau):
                def ksacc(j, a):
                    ka, sa = a
                    v = cval_v[pl.ds(j * L, L)]
                    m = v > tau
                    return (ka + m.astype(jnp.float32),
                            sa + jnp.where(m, v, 0.0))

                ka, sa = lax.fori_loop(0, nvc, ksacc,
                                       (_splat_f(0.0), _splat_f(0.0)))
                return (_sum_splat(sa) - 1.0) / _sum_splat(ka)

            return lax.fori_loop(0, N_MICHELOT, michelot, lo)

        def tau_full():
            def bisect(_, lohi):
                lo, hi = lohi
                mid = 0.5 * (lo + hi)

                def facc(j, a):
                    return a + jnp.maximum(row_v[pl.ds(j * L, L)] - mid, 0.0)

                f = _sum_splat(lax.fori_loop(0, NCHUNK, facc, _splat_f(0.0)))
                ge = f >= 1.0
                return jnp.where(ge, mid, lo), jnp.where(ge, hi, mid)

            lo, _ = lax.fori_loop(0, N_BISECT_FULL, bisect,
                                  (mx_v - 1.0, mx_v))
            return lo

        return lax.cond(ok, tau_fast, tau_full)

    def scatter_list(cval_v, cidx_v, nvc, tau_v, value_from):
        def one(j, carry):
            v = cval_v[pl.ds(j * L, L)]
            idx = cidx_v[pl.ds(j * L, L)]
            m = v > tau_v
            plsc.store_scatter(out_v, [idx], value_from(v), mask=m)
            return carry

        lax.fori_loop(0, nvc, one, 0)

    def restore_zeros(prev):
        h, cval_v, cidx_v, nvc, tau_v, ok = prev
        h.wait()

        @pl.when(ok)
        def _():
            scatter_list(cval_v, cidx_v, nvc, tau_v, lambda v: _splat_f(0.0))

        @pl.when(jnp.logical_not(ok))
        def _():
            lax.fori_loop(0, NCHUNK // ZUNROLL, zero_all, 0)

    ibufs = [(row0_v, sem0), (row1_v, sem1)]
    cbufs = [(cval0_v, cidx0_v), (cval1_v, cidx1_v)]
    handles = [pltpu.async_copy(x_hbm.at[base], row0_v, sem0), None]
    pending = None
    for i in range(ROWS_PER_W):
        row_v, _ = ibufs[i % 2]
        cval_v, cidx_v = cbufs[i % 2]
        handles[i % 2].wait()
        if i + 1 < ROWS_PER_W:
            nbuf, nsem = ibufs[(i + 1) % 2]
            handles[(i + 1) % 2] = pltpu.async_copy(
                x_hbm.at[base + i + 1], nbuf, nsem)

        with jax.named_scope("ph_filter"):
            ok, nvc, mx_v = filter_row(row_v, cval_v, cidx_v)
        with jax.named_scope("ph_tau"):
            tau_v = solve_tau(row_v, cval_v, ok, nvc, mx_v)

        if pending is not None:
            restore_zeros(pending)

        @pl.when(ok)
        def _():
            scatter_list(cval_v, cidx_v, nvc, tau_v, lambda v: v - tau_v)

        @pl.when(jnp.logical_not(ok))
        def _():
            def dense(j, carry):
                v = row_v[pl.ds(j * L, L)]
                out_v[pl.ds(j * L, L)] = jnp.maximum(v - tau_v, 0.0)
                return carry

            lax.fori_loop(0, NCHUNK, dense, 0)

        h = pltpu.async_copy(out_v, o_hbm.at[base + i], osem)
        pending = (h, cval_v, cidx_v, nvc, tau_v, ok)

    pending[0].wait()


@jax.jit
def kernel(scores):
    f = functools.partial(
        pl.kernel,
        mesh=_mesh,
        out_type=jax.ShapeDtypeStruct((ROWS, N), jnp.float32),
        compiler_params=pltpu.CompilerParams(needs_layout_passes=False),
        scratch_types=[
            pltpu.VMEM((N,), jnp.float32),
            pltpu.VMEM((N,), jnp.float32),
            pltpu.VMEM((N,), jnp.float32),
            pltpu.VMEM((NCHUNK,), jnp.float32),
            pltpu.VMEM((UCAP,), jnp.int32),
            pltpu.VMEM((CAP,), jnp.float32),
            pltpu.VMEM((CAP,), jnp.int32),
            pltpu.VMEM((CAP,), jnp.float32),
            pltpu.VMEM((CAP,), jnp.int32),
            pltpu.SemaphoreType.DMA,
            pltpu.SemaphoreType.DMA,
            pltpu.SemaphoreType.DMA,
        ],
    )(_sc_body)
    return f(scores)


# speedup vs baseline: 56.2523x; 1.0368x over previous
import functools

import jax
import jax.numpy as jnp
from jax import lax
from jax.experimental import pallas as pl
from jax.experimental.pallas import tpu as pltpu
from jax.experimental.pallas import tpu_sc as plsc

ROWS = 128
N = 32768
L = 16
NCHUNK = N // L
NGVEC = NCHUNK // L
NW = 32
ROWS_PER_W = ROWS // NW

UCAP = 1024
CAP = 4096
NEG = -1e30
ZUNROLL = 8

N_BISECT = 12
N_MICHELOT = 3
N_BISECT_FULL = 42

_mesh = plsc.VectorSubcoreMesh(core_axis_name="c", subcore_axis_name="s")


def _iota():
    return lax.iota(jnp.int32, L)


def _splat_f(x):
    return jnp.full((L,), x, jnp.float32)


def _tree_max(vs):
    while len(vs) > 1:
        vs = [jnp.maximum(a, b) for a, b in zip(vs[::2], vs[1::2])]
    return vs[0]


def _sum_splat(av):
    return jnp.full((L,), jnp.sum(av), jnp.float32)


def _sc_body(x_hbm, o_hbm, row0_v, row1_v, out_v, gmax_v, unit_v,
             cval0_v, cidx0_v, cval1_v, cidx1_v, sem0, sem1, osem):
    wid = lax.axis_index("s") * 2 + lax.axis_index("c")
    base = wid * ROWS_PER_W

    def zero_all(j, carry):
        for u in range(ZUNROLL):
            out_v[pl.ds((j * ZUNROLL + u) * L, L)] = _splat_f(0.0)
        return carry

    lax.fori_loop(0, NCHUNK // ZUNROLL, zero_all, 0)

    def filter_row(row_v, cval_v, cidx_v):
        def gmax_one(g2, carry):
            for u in range(2):
                g = g2 * 2 + u
                vs = [row_v[pl.ds((g + NGVEC * k) * L, L)] for k in range(L)]
                gmax_v[pl.ds(g * L, L)] = _tree_max(vs)
            return carry

        lax.fori_loop(0, NGVEC // 2, gmax_one, 0)

        def max_red(j, acc):
            vs = [gmax_v[pl.ds((j * 8 + u) * L, L)] for u in range(8)]
            return jnp.maximum(acc, _tree_max(vs))

        gacc = lax.fori_loop(0, NGVEC // 8, max_red, _splat_f(NEG))
        mx_v = jnp.full((L,), jnp.max(gacc), jnp.float32)
        thr_v = mx_v - 1.0

        def filt_units(q, offv):
            ms = [gmax_v[pl.ds((q * 4 + u) * L, L)] for u in range(4)]
            sels = [m > thr_v for m in ms]
            for u in range(4):
                sel = sels[u]
                pos = offv + plsc.cumsum(sel.astype(jnp.int32)) - 1
                keep = sel & (pos < UCAP)
                plsc.store_scatter(unit_v, [pos],
                                   (q * 4 + u) * L + _iota(), mask=keep)
                offv = offv + plsc.all_reduce_population_count(sel)
            return offv

        offv = lax.fori_loop(0, NGVEC // 4, filt_units,
                             jnp.zeros((L,), jnp.int32))
        nunit = jnp.max(offv)
        nunit_v = jnp.full((L,), nunit, jnp.int32)

        nub = (jnp.minimum(nunit, UCAP) + (L - 1)) // L

        def gather_units(b, offv):
            lane_ok = (b * L + _iota()) < nunit_v
            uvec = jnp.where(lane_ok, unit_v[pl.ds(b * L, L)], 0)

            def one_k(k, offv):
                idx = uvec + NCHUNK * k
                v = plsc.load_gather(row_v, [idx])
                sel = (v > thr_v) & lane_ok
                pos = offv + plsc.cumsum(sel.astype(jnp.int32)) - 1
                keep = sel & (pos < CAP)
                plsc.store_scatter(cval_v, [pos], v, mask=keep)
                plsc.store_scatter(cidx_v, [pos], idx, mask=keep)
                return offv + plsc.all_reduce_population_count(sel)

            return lax.fori_loop(0, L, one_k, offv)

        offv = lax.fori_loop(0, nub, gather_units, jnp.zeros((L,), jnp.int32))
        ncand = jnp.max(offv)
        ok = (ncand <= CAP) & (nunit <= UCAP)

        padpos = offv + _iota()
        plsc.store_scatter(cval_v, [padpos], _splat_f(NEG),
                           mask=padpos < CAP)

        nvc = (jnp.minimum(ncand, CAP) + (L - 1)) // L
        return ok, nvc, mx_v

    def solve_tau(row_v, cval_v, ok, nvc, mx_v):
        def ge_one(av):
            m = plsc.cumsum(av) >= 1.0
            return plsc.all_reduce_population_count(m) > 0

        def tau_fast():
            def bisect(_, lohi):
                lo, hi = lohi
                mid = 0.5 * (lo + hi)

                def facc(j, a):
                    return a + jnp.maximum(cval_v[pl.ds(j * L, L)] - mid, 0.0)

                ge = ge_one(lax.fori_loop(0, nvc, facc, _splat_f(0.0)))
                return jnp.where(ge, mid, lo), jnp.where(ge, hi, mid)

            lo, _ = lax.fori_loop(0, N_BISECT, bisect, (mx_v - 1.0, mx_v))

            def michelot(_, tau):
                def ksacc(j, a):
                    kv, sa = a
                    v = cval_v[pl.ds(j * L, L)]
                    m = v > tau
                    return (kv + plsc.all_reduce_population_count(m),
                            sa + jnp.where(m, v, 0.0))

                kv, sa = lax.fori_loop(
                    0, nvc, ksacc, (jnp.zeros((L,), jnp.int32), _splat_f(0.0)))
                return (_sum_splat(sa) - 1.0) / kv.astype(jnp.float32)

            return lax.fori_loop(0, N_MICHELOT, michelot, lo)

        def tau_full():
            def bisect(_, lohi):
                lo, hi = lohi
                mid = 0.5 * (lo + hi)

                def facc(j, a):
                    return a + jnp.maximum(row_v[pl.ds(j * L, L)] - mid, 0.0)

                ge = ge_one(lax.fori_loop(0, NCHUNK, facc, _splat_f(0.0)))
                return jnp.where(ge, mid, lo), jnp.where(ge, hi, mid)

            lo, _ = lax.fori_loop(0, N_BISECT_FULL, bisect,
                                  (mx_v - 1.0, mx_v))
            return lo

        return lax.cond(ok, tau_fast, tau_full)

    def scatter_list(cval_v, cidx_v, nvc, tau_v, value_from):
        def one(j, carry):
            v = cval_v[pl.ds(j * L, L)]
            idx = cidx_v[pl.ds(j * L, L)]
            m = v > tau_v
            plsc.store_scatter(out_v, [idx], value_from(v), mask=m)
            return carry

        lax.fori_loop(0, nvc, one, 0)

    def restore_zeros(prev):
        h, cval_v, cidx_v, nvc, tau_v, ok = prev
        h.wait()

        @pl.when(ok)
        def _():
            scatter_list(cval_v, cidx_v, nvc, tau_v, lambda v: _splat_f(0.0))

        @pl.when(jnp.logical_not(ok))
        def _():
            lax.fori_loop(0, NCHUNK // ZUNROLL, zero_all, 0)

    ibufs = [(row0_v, sem0), (row1_v, sem1)]
    cbufs = [(cval0_v, cidx0_v), (cval1_v, cidx1_v)]
    handles = [pltpu.async_copy(x_hbm.at[base], row0_v, sem0), None]
    pending = None
    for i in range(ROWS_PER_W):
        row_v, _ = ibufs[i % 2]
        cval_v, cidx_v = cbufs[i % 2]
        handles[i % 2].wait()
        if i + 1 < ROWS_PER_W:
            nbuf, nsem = ibufs[(i + 1) % 2]
            handles[(i + 1) % 2] = pltpu.async_copy(
                x_hbm.at[base + i + 1], nbuf, nsem)

        ok, nvc, mx_v = filter_row(row_v, cval_v, cidx_v)
        tau_v = solve_tau(row_v, cval_v, ok, nvc, mx_v)

        if pending is not None:
            restore_zeros(pending)

        @pl.when(ok)
        def _():
            scatter_list(cval_v, cidx_v, nvc, tau_v, lambda v: v - tau_v)

        @pl.when(jnp.logical_not(ok))
        def _():
            def dense(j, carry):
                v = row_v[pl.ds(j * L, L)]
                out_v[pl.ds(j * L, L)] = jnp.maximum(v - tau_v, 0.0)
                return carry

            lax.fori_loop(0, NCHUNK, dense, 0)

        h = pltpu.async_copy(out_v, o_hbm.at[base + i], osem)
        pending = (h, cval_v, cidx_v, nvc, tau_v, ok)

    pending[0].wait()


@jax.jit
def kernel(scores):
    f = functools.partial(
        pl.kernel,
        mesh=_mesh,
        out_type=jax.ShapeDtypeStruct((ROWS, N), jnp.float32),
        compiler_params=pltpu.CompilerParams(needs_layout_passes=False),
        scratch_types=[
            pltpu.VMEM((N,), jnp.float32),
            pltpu.VMEM((N,), jnp.float32),
            pltpu.VMEM((N,), jnp.float32),
            pltpu.VMEM((NCHUNK,), jnp.float32),
            pltpu.VMEM((UCAP,), jnp.int32),
            pltpu.VMEM((CAP,), jnp.float32),
            pltpu.VMEM((CAP,), jnp.int32),
            pltpu.VMEM((CAP,), jnp.float32),
            pltpu.VMEM((CAP,), jnp.int32),
            pltpu.SemaphoreType.DMA,
            pltpu.SemaphoreType.DMA,
            pltpu.SemaphoreType.DMA,
        ],
    )(_sc_body)
    return f(scores)


# speedup vs baseline: 57.6526x; 1.0249x over previous
import functools

import jax
import jax.numpy as jnp
from jax import lax
from jax.experimental import pallas as pl
from jax.experimental.pallas import tpu as pltpu
from jax.experimental.pallas import tpu_sc as plsc

ROWS = 128
N = 32768
L = 16
NCHUNK = N // L
NGVEC = NCHUNK // L
NW = 32
ROWS_PER_W = ROWS // NW

UCAP = 1024
CAP = 4096
NEG = -1e30
ZUNROLL = 8

N_BISECT = 8
N_MICHELOT = 3
N_BISECT_FULL = 42

_mesh = plsc.VectorSubcoreMesh(core_axis_name="c", subcore_axis_name="s")


def _iota():
    return lax.iota(jnp.int32, L)


def _splat_f(x):
    return jnp.full((L,), x, jnp.float32)


def _tree_max(vs):
    while len(vs) > 1:
        vs = [jnp.maximum(a, b) for a, b in zip(vs[::2], vs[1::2])]
    return vs[0]


def _sum_splat(av):
    return jnp.full((L,), jnp.sum(av), jnp.float32)


def _sc_body(x_hbm, o_hbm, row0_v, row1_v, out_v, gmax_v, unit_v,
             cval0_v, cidx0_v, cval1_v, cidx1_v, sem0, sem1, osem):
    wid = lax.axis_index("s") * 2 + lax.axis_index("c")
    base = wid * ROWS_PER_W

    def zero_all(j, carry):
        for u in range(ZUNROLL):
            out_v[pl.ds((j * ZUNROLL + u) * L, L)] = _splat_f(0.0)
        return carry

    lax.fori_loop(0, NCHUNK // ZUNROLL, zero_all, 0)

    def filter_row(row_v, cval_v, cidx_v):
        def gmax_one(g2, carry):
            for u in range(2):
                g = g2 * 2 + u
                vs = [row_v[pl.ds((g + NGVEC * k) * L, L)] for k in range(L)]
                gmax_v[pl.ds(g * L, L)] = _tree_max(vs)
            return carry

        lax.fori_loop(0, NGVEC // 2, gmax_one, 0)

        def max_red(j, acc):
            vs = [gmax_v[pl.ds((j * 8 + u) * L, L)] for u in range(8)]
            return jnp.maximum(acc, _tree_max(vs))

        gacc = lax.fori_loop(0, NGVEC // 8, max_red, _splat_f(NEG))
        mx_v = jnp.full((L,), jnp.max(gacc), jnp.float32)
        thr_v = mx_v - 1.0

        def filt_units(q, offv):
            ms = [gmax_v[pl.ds((q * 8 + u) * L, L)] for u in range(8)]
            sels = [m > thr_v for m in ms]
            for u in range(8):
                sel = sels[u]
                pos = offv + plsc.cumsum(sel.astype(jnp.int32)) - 1
                keep = sel & (pos < UCAP)
                plsc.store_scatter(unit_v, [pos],
                                   (q * 8 + u) * L + _iota(), mask=keep)
                offv = offv + plsc.all_reduce_population_count(sel)
            return offv

        offv = lax.fori_loop(0, NGVEC // 8, filt_units,
                             jnp.zeros((L,), jnp.int32))
        nunit = jnp.max(offv)
        nunit_v = jnp.full((L,), nunit, jnp.int32)

        nub = (jnp.minimum(nunit, UCAP) + (L - 1)) // L

        def gather_units(b, offv):
            lane_ok = (b * L + _iota()) < nunit_v
            uvec = jnp.where(lane_ok, unit_v[pl.ds(b * L, L)], 0)

            def one_k(k4, offv):
                for kk in range(4):
                    idx = uvec + NCHUNK * (k4 * 4 + kk)
                    v = plsc.load_gather(row_v, [idx])
                    sel = (v > thr_v) & lane_ok
                    pos = offv + plsc.cumsum(sel.astype(jnp.int32)) - 1
                    keep = sel & (pos < CAP)
                    plsc.store_scatter(cval_v, [pos], v, mask=keep)
                    plsc.store_scatter(cidx_v, [pos], idx, mask=keep)
                    offv = offv + plsc.all_reduce_population_count(sel)
                return offv

            return lax.fori_loop(0, L // 4, one_k, offv)

        offv = lax.fori_loop(0, nub, gather_units, jnp.zeros((L,), jnp.int32))
        ncand = jnp.max(offv)
        ok = (ncand <= CAP) & (nunit <= UCAP)

        padpos = offv + _iota()
        plsc.store_scatter(cval_v, [padpos], _splat_f(NEG),
                           mask=padpos < CAP)

        nvc = (jnp.minimum(ncand, CAP) + (L - 1)) // L
        return ok, nvc, mx_v

    def solve_tau(row_v, cval_v, ok, nvc, mx_v):
        def ge_one(av):
            m = plsc.cumsum(av) >= 1.0
            return plsc.all_reduce_population_count(m) > 0

        def tau_fast():
            def bisect(_, lohi):
                lo, hi = lohi
                mid = 0.5 * (lo + hi)

                def facc(j, a):
                    return a + jnp.maximum(cval_v[pl.ds(j * L, L)] - mid, 0.0)

                ge = ge_one(lax.fori_loop(0, nvc, facc, _splat_f(0.0)))
                return jnp.where(ge, mid, lo), jnp.where(ge, hi, mid)

            lo, _ = lax.fori_loop(0, N_BISECT, bisect, (mx_v - 1.0, mx_v))

            def michelot(_, tau):
                def ksacc(j, a):
                    kv, sa = a
                    v = cval_v[pl.ds(j * L, L)]
                    m = v > tau
                    return (kv + plsc.all_reduce_population_count(m),
                            sa + jnp.where(m, v, 0.0))

                kv, sa = lax.fori_loop(
                    0, nvc, ksacc, (jnp.zeros((L,), jnp.int32), _splat_f(0.0)))
                return (_sum_splat(sa) - 1.0) / kv.astype(jnp.float32)

            return lax.fori_loop(0, N_MICHELOT, michelot, lo)

        def tau_full():
            def bisect(_, lohi):
                lo, hi = lohi
                mid = 0.5 * (lo + hi)

                def facc(j, a):
                    return a + jnp.maximum(row_v[pl.ds(j * L, L)] - mid, 0.0)

                ge = ge_one(lax.fori_loop(0, NCHUNK, facc, _splat_f(0.0)))
                return jnp.where(ge, mid, lo), jnp.where(ge, hi, mid)

            lo, _ = lax.fori_loop(0, N_BISECT_FULL, bisect,
                                  (mx_v - 1.0, mx_v))
            return lo

        return lax.cond(ok, tau_fast, tau_full)

    def scatter_list(cval_v, cidx_v, nvc, tau_v, value_from):
        def one(j, carry):
            v = cval_v[pl.ds(j * L, L)]
            idx = cidx_v[pl.ds(j * L, L)]
            m = v > tau_v
            plsc.store_scatter(out_v, [idx], value_from(v), mask=m)
            return carry

        lax.fori_loop(0, nvc, one, 0)

    def restore_zeros(prev):
        h, cval_v, cidx_v, nvc, tau_v, ok = prev
        h.wait()

        @pl.when(ok)
        def _():
            scatter_list(cval_v, cidx_v, nvc, tau_v, lambda v: _splat_f(0.0))

        @pl.when(jnp.logical_not(ok))
        def _():
            lax.fori_loop(0, NCHUNK // ZUNROLL, zero_all, 0)

    ibufs = [(row0_v, sem0), (row1_v, sem1)]
    cbufs = [(cval0_v, cidx0_v), (cval1_v, cidx1_v)]
    handles = [pltpu.async_copy(x_hbm.at[base], row0_v, sem0), None]
    pending = None
    for i in range(ROWS_PER_W):
        row_v, _ = ibufs[i % 2]
        cval_v, cidx_v = cbufs[i % 2]
        handles[i % 2].wait()
        if i + 1 < ROWS_PER_W:
            nbuf, nsem = ibufs[(i + 1) % 2]
            handles[(i + 1) % 2] = pltpu.async_copy(
                x_hbm.at[base + i + 1], nbuf, nsem)

        ok, nvc, mx_v = filter_row(row_v, cval_v, cidx_v)
        tau_v = solve_tau(row_v, cval_v, ok, nvc, mx_v)

        if pending is not None:
            restore_zeros(pending)

        @pl.when(ok)
        def _():
            scatter_list(cval_v, cidx_v, nvc, tau_v, lambda v: v - tau_v)

        @pl.when(jnp.logical_not(ok))
        def _():
            def dense(j, carry):
                v = row_v[pl.ds(j * L, L)]
                out_v[pl.ds(j * L, L)] = jnp.maximum(v - tau_v, 0.0)
                return carry

            lax.fori_loop(0, NCHUNK, dense, 0)

        h = pltpu.async_copy(out_v, o_hbm.at[base + i], osem)
        pending = (h, cval_v, cidx_v, nvc, tau_v, ok)

    pending[0].wait()


@jax.jit
def kernel(scores):
    f = functools.partial(
        pl.kernel,
        mesh=_mesh,
        out_type=jax.ShapeDtypeStruct((ROWS, N), jnp.float32),
        compiler_params=pltpu.CompilerParams(needs_layout_passes=False),
        scratch_types=[
            pltpu.VMEM((N,), jnp.float32),
            pltpu.VMEM((N,), jnp.float32),
            pltpu.VMEM((N,), jnp.float32),
            pltpu.VMEM((NCHUNK,), jnp.float32),
            pltpu.VMEM((UCAP,), jnp.int32),
            pltpu.VMEM((CAP,), jnp.float32),
            pltpu.VMEM((CAP,), jnp.int32),
            pltpu.VMEM((CAP,), jnp.float32),
            pltpu.VMEM((CAP,), jnp.int32),
            pltpu.SemaphoreType.DMA,
            pltpu.SemaphoreType.DMA,
            pltpu.SemaphoreType.DMA,
        ],
    )(_sc_body)
    return f(scores)


# speedup vs baseline: 59.2012x; 1.0269x over previous
import functools

import jax
import jax.numpy as jnp
from jax import lax
from jax.experimental import pallas as pl
from jax.experimental.pallas import tpu as pltpu
from jax.experimental.pallas import tpu_sc as plsc

ROWS = 128
N = 32768
L = 16
NCHUNK = N // L
NGVEC = NCHUNK // L
NW = 32
ROWS_PER_W = ROWS // NW

UCAP = 1024
CAP = 4096
NEG = -1e30
ZUNROLL = 8

N_BISECT = 8
N_MICHELOT = 3
N_BISECT_FULL = 42

_mesh = plsc.VectorSubcoreMesh(core_axis_name="c", subcore_axis_name="s")


def _iota():
    return lax.iota(jnp.int32, L)


def _splat_f(x):
    return jnp.full((L,), x, jnp.float32)


def _tree_max(vs):
    while len(vs) > 1:
        vs = [jnp.maximum(a, b) for a, b in zip(vs[::2], vs[1::2])]
    return vs[0]


def _sum_splat(av):
    return jnp.full((L,), jnp.sum(av), jnp.float32)


def _sc_body(x_hbm, o_hbm, row0_v, row1_v, out_v, gmax_v, unit_v,
             cval0_v, cidx0_v, cval1_v, cidx1_v, sem0, sem1, osem):
    wid = lax.axis_index("s") * 2 + lax.axis_index("c")
    base = wid * ROWS_PER_W

    def zero_all(j, carry):
        for u in range(ZUNROLL):
            out_v[pl.ds((j * ZUNROLL + u) * L, L)] = _splat_f(0.0)
        return carry

    first_h = pltpu.async_copy(x_hbm.at[base], row0_v, sem0)

    lax.fori_loop(0, NCHUNK // ZUNROLL, zero_all, 0)

    def filter_row(row_v, cval_v, cidx_v):
        def gmax_one(g2, carry):
            for u in range(2):
                g = g2 * 2 + u
                vs = [row_v[pl.ds((g + NGVEC * k) * L, L)] for k in range(L)]
                gmax_v[pl.ds(g * L, L)] = _tree_max(vs)
            return carry

        lax.fori_loop(0, NGVEC // 2, gmax_one, 0)

        def max_red(j, acc):
            vs = [gmax_v[pl.ds((j * 8 + u) * L, L)] for u in range(8)]
            return jnp.maximum(acc, _tree_max(vs))

        gacc = lax.fori_loop(0, NGVEC // 8, max_red, _splat_f(NEG))
        mx_v = jnp.full((L,), jnp.max(gacc), jnp.float32)
        thr_v = mx_v - 1.0

        def filt_units(q, offv):
            ms = [gmax_v[pl.ds((q * 8 + u) * L, L)] for u in range(8)]
            sels = [m > thr_v for m in ms]
            for u in range(8):
                sel = sels[u]
                pos = offv + plsc.cumsum(sel.astype(jnp.int32)) - 1
                keep = sel & (pos < UCAP)
                plsc.store_scatter(unit_v, [pos],
                                   (q * 8 + u) * L + _iota(), mask=keep)
                offv = offv + plsc.all_reduce_population_count(sel)
            return offv

        offv = lax.fori_loop(0, NGVEC // 8, filt_units,
                             jnp.zeros((L,), jnp.int32))
        nunit = jnp.max(offv)
        nunit_v = jnp.full((L,), nunit, jnp.int32)

        nub = (jnp.minimum(nunit, UCAP) + (L - 1)) // L

        def gather_units(b, offv):
            lane_ok = (b * L + _iota()) < nunit_v
            uvec = jnp.where(lane_ok, unit_v[pl.ds(b * L, L)], 0)

            def one_k(k4, offv):
                for kk in range(4):
                    idx = uvec + NCHUNK * (k4 * 4 + kk)
                    v = plsc.load_gather(row_v, [idx])
                    sel = (v > thr_v) & lane_ok
                    pos = offv + plsc.cumsum(sel.astype(jnp.int32)) - 1
                    keep = sel & (pos < CAP)
                    plsc.store_scatter(cval_v, [pos], v, mask=keep)
                    plsc.store_scatter(cidx_v, [pos], idx, mask=keep)
                    offv = offv + plsc.all_reduce_population_count(sel)
                return offv

            return lax.fori_loop(0, L // 4, one_k, offv)

        offv = lax.fori_loop(0, nub, gather_units, jnp.zeros((L,), jnp.int32))
        ncand = jnp.max(offv)
        ok = (ncand <= CAP) & (nunit <= UCAP)

        padpos = offv + _iota()
        plsc.store_scatter(cval_v, [padpos], _splat_f(NEG),
                           mask=padpos < CAP)

        nvc = (jnp.minimum(ncand, CAP) + (L - 1)) // L
        return ok, nvc, mx_v

    def solve_tau(row_v, cval_v, ok, nvc, mx_v):
        def ge_one(av):
            m = plsc.cumsum(av) >= 1.0
            return plsc.all_reduce_population_count(m) > 0

        def tau_fast():
            def bisect(_, lohi):
                lo, hi = lohi
                mid = 0.5 * (lo + hi)

                def facc(j, a):
                    return a + jnp.maximum(cval_v[pl.ds(j * L, L)] - mid, 0.0)

                ge = ge_one(lax.fori_loop(0, nvc, facc, _splat_f(0.0)))
                return jnp.where(ge, mid, lo), jnp.where(ge, hi, mid)

            lo, _ = lax.fori_loop(0, N_BISECT, bisect, (mx_v - 1.0, mx_v))

            def michelot(_, tau):
                def ksacc(j, a):
                    kv, sa = a
                    v = cval_v[pl.ds(j * L, L)]
                    m = v > tau
                    return (kv + plsc.all_reduce_population_count(m),
                            sa + jnp.where(m, v, 0.0))

                kv, sa = lax.fori_loop(
                    0, nvc, ksacc, (jnp.zeros((L,), jnp.int32), _splat_f(0.0)))
                return (_sum_splat(sa) - 1.0) / kv.astype(jnp.float32)

            return lax.fori_loop(0, N_MICHELOT, michelot, lo)

        def tau_full():
            def bisect(_, lohi):
                lo, hi = lohi
                mid = 0.5 * (lo + hi)

                def facc(j, a):
                    return a + jnp.maximum(row_v[pl.ds(j * L, L)] - mid, 0.0)

                ge = ge_one(lax.fori_loop(0, NCHUNK, facc, _splat_f(0.0)))
                return jnp.where(ge, mid, lo), jnp.where(ge, hi, mid)

            lo, _ = lax.fori_loop(0, N_BISECT_FULL, bisect,
                                  (mx_v - 1.0, mx_v))
            return lo

        return lax.cond(ok, tau_fast, tau_full)

    def scatter_list(cval_v, cidx_v, nvc, tau_v, value_from):
        def one(j, carry):
            v = cval_v[pl.ds(j * L, L)]
            idx = cidx_v[pl.ds(j * L, L)]
            m = v > tau_v
            plsc.store_scatter(out_v, [idx], value_from(v), mask=m)
            return carry

        lax.fori_loop(0, nvc, one, 0)

    def restore_zeros(prev):
        h, cval_v, cidx_v, nvc, tau_v, ok = prev
        h.wait()

        @pl.when(ok)
        def _():
            scatter_list(cval_v, cidx_v, nvc, tau_v, lambda v: _splat_f(0.0))

        @pl.when(jnp.logical_not(ok))
        def _():
            lax.fori_loop(0, NCHUNK // ZUNROLL, zero_all, 0)

    ibufs = [(row0_v, sem0), (row1_v, sem1)]
    cbufs = [(cval0_v, cidx0_v), (cval1_v, cidx1_v)]
    handles = [first_h, None]
    pending = None
    for i in range(ROWS_PER_W):
        row_v, _ = ibufs[i % 2]
        cval_v, cidx_v = cbufs[i % 2]
        handles[i % 2].wait()
        if i + 1 < ROWS_PER_W:
            nbuf, nsem = ibufs[(i + 1) % 2]
            handles[(i + 1) % 2] = pltpu.async_copy(
                x_hbm.at[base + i + 1], nbuf, nsem)

        ok, nvc, mx_v = filter_row(row_v, cval_v, cidx_v)
        tau_v = solve_tau(row_v, cval_v, ok, nvc, mx_v)

        if pending is not None:
            restore_zeros(pending)

        @pl.when(ok)
        def _():
            scatter_list(cval_v, cidx_v, nvc, tau_v, lambda v: v - tau_v)

        @pl.when(jnp.logical_not(ok))
        def _():
            def dense(j, carry):
                v = row_v[pl.ds(j * L, L)]
                out_v[pl.ds(j * L, L)] = jnp.maximum(v - tau_v, 0.0)
                return carry

            lax.fori_loop(0, NCHUNK, dense, 0)

        h = pltpu.async_copy(out_v, o_hbm.at[base + i], osem)
        pending = (h, cval_v, cidx_v, nvc, tau_v, ok)

    pending[0].wait()


@jax.jit
def kernel(scores):
    f = functools.partial(
        pl.kernel,
        mesh=_mesh,
        out_type=jax.ShapeDtypeStruct((ROWS, N), jnp.float32),
        compiler_params=pltpu.CompilerParams(needs_layout_passes=False),
        scratch_types=[
            pltpu.VMEM((N,), jnp.float32),
            pltpu.VMEM((N,), jnp.float32),
            pltpu.VMEM((N,), jnp.float32),
            pltpu.VMEM((NCHUNK,), jnp.float32),
            pltpu.VMEM((UCAP,), jnp.int32),
            pltpu.VMEM((CAP,), jnp.float32),
            pltpu.VMEM((CAP,), jnp.int32),
            pltpu.VMEM((CAP,), jnp.float32),
            pltpu.VMEM((CAP,), jnp.int32),
            pltpu.SemaphoreType.DMA,
            pltpu.SemaphoreType.DMA,
            pltpu.SemaphoreType.DMA,
        ],
    )(_sc_body)
    return f(scores)
